# baseline (speedup 1.0000x reference)
import functools

import jax
import jax.numpy as jnp
from jax import lax
from jax.experimental import pallas as pl
from jax.experimental.pallas import tpu as pltpu
from jax.experimental.pallas import tpu_sc as plsc

BATCH = 16384
EMB = 16
NUM_NUM = 16
UNITS = 128
GCOLS = 3 * (EMB + 1)

_NC = 2
_NS = 16
_NW = _NC * _NS
_L = 16
_NCHUNK = 2
_CB = BATCH // _NCHUNK
_BPW = _CB // _NW
_IPR = _BPW // 128


def _sc_gather_body(idx_hbm, t0, t1, t2, b0, b1, b2, g_hbm,
                    idx_v, t0v, t1v, t2v, b0v, b1v, b2v, g_v):
    wid = lax.axis_index("s") * _NC + lax.axis_index("c")
    pltpu.sync_copy(idx_hbm.at[wid], idx_v)
    pltpu.sync_copy(t0, t0v)
    pltpu.sync_copy(t1, t1v)
    pltpu.sync_copy(t2, t2v)
    pltpu.sync_copy(b0, b0v)
    pltpu.sync_copy(b1, b1v)
    pltpu.sync_copy(b2, b2v)

    tables = ((t0v, b0v, 0), (t1v, b1v, EMB + 1), (t2v, b2v, 2 * (EMB + 1)))
    for kk in range(_IPR):
        for c in range(8):
            k = kk * 8 + c
            for t, (tv, bv, row0) in enumerate(tables):
                iv = idx_v[t * _IPR + kk, pl.ds(c * _L, _L)]
                lin = iv * EMB
                for j in range(EMB):
                    lj = lin + j
                    vals = plsc.load_gather(
                        tv, [lax.shift_right_logical(lj, 7),
                             lax.bitwise_and(lj, 127)])
                    g_v[row0 + j, pl.ds(k * _L, _L)] = vals
                bvals = plsc.load_gather(
                    bv, [lax.shift_right_logical(iv, 7),
                         lax.bitwise_and(iv, 127)])
                g_v[row0 + EMB, pl.ds(k * _L, _L)] = bvals

    pltpu.sync_copy(g_v, g_hbm.at[:, pl.ds(wid * _BPW, _BPW)])


def _sc_gather(idxw, t0, t1, t2, b0, b1, b2):
    mesh = plsc.VectorSubcoreMesh(core_axis_name="c", subcore_axis_name="s")
    f = functools.partial(
        pl.kernel,
        mesh=mesh,
        out_type=jax.ShapeDtypeStruct((GCOLS, _CB), jnp.float32),
        scratch_types=[
            pltpu.VMEM((3 * _IPR, 128), jnp.int32),
            pltpu.VMEM(t0.shape, jnp.float32),
            pltpu.VMEM(t1.shape, jnp.float32),
            pltpu.VMEM(t2.shape, jnp.float32),
            pltpu.VMEM(b0.shape, jnp.float32),
            pltpu.VMEM(b1.shape, jnp.float32),
            pltpu.VMEM(b2.shape, jnp.float32),
            pltpu.VMEM((GCOLS, _BPW), jnp.float32),
        ],
        compiler_params=pltpu.CompilerParams(needs_layout_passes=False),
    )(_sc_gather_body)
    return f(idxw, t0, t1, t2, b0, b1, b2)


def _mlp_body(inp_ref, g_ref, a1, w1n, c1, w2, c2, w3, c3, w4, c4, w5, c5,
              out_ref):
    dot = functools.partial(jnp.dot, preferred_element_type=jnp.float32)
    num = inp_ref[:, 3:3 + NUM_NUM]
    h = lax.dot_general(g_ref[...], a1[...], (((0,), (0,)), ((), ())),
                        preferred_element_type=jnp.float32)
    h = h + dot(num, w1n[...]) + c1[...]
    h = jnp.maximum(h, 0.0)
    h = jnp.maximum(dot(h, w2[...]) + c2[...], 0.0)
    h = jnp.maximum(dot(h, w3[...]) + c3[...], 0.0)
    h = jnp.maximum(dot(h, w4[...]) + c4[...], 0.0)
    logits = dot(h, w5[...]) + c5[...]
    m = jnp.max(logits, axis=-1, keepdims=True)
    e = jnp.exp(logits - m)
    out_ref[...] = e / jnp.sum(e, axis=-1, keepdims=True)


def _full(shape):
    return pl.BlockSpec(shape, lambda i: (0, 0))


def _mlp(inputs, g, a1, w1n, c1, w2, c2, w3, c3, w4, c4, w5, c5, block_b):
    nlab = w5.shape[1]
    nb = inputs.shape[0]
    grid = (nb // block_b,)
    in_specs = [
        pl.BlockSpec((block_b, inputs.shape[1]), lambda i: (i, 0)),
        pl.BlockSpec((GCOLS, block_b), lambda i: (0, i)),
        _full(a1.shape), _full(w1n.shape), _full(c1.shape),
        _full(w2.shape), _full(c2.shape), _full(w3.shape), _full(c3.shape),
        _full(w4.shape), _full(c4.shape), _full(w5.shape), _full(c5.shape),
    ]
    return pl.pallas_call(
        _mlp_body,
        grid=grid,
        in_specs=in_specs,
        out_specs=pl.BlockSpec((block_b, nlab), lambda i: (i, 0)),
        out_shape=jax.ShapeDtypeStruct((nb, nlab), jnp.float32),
        compiler_params=pltpu.CompilerParams(
            dimension_semantics=("arbitrary",)),
    )(inputs, g, a1, w1n, c1, w2, c2, w3, c3, w4, c4, w5, c5)


def kernel(inputs, speed_emb, speed_bias, oneway_emb, oneway_bias, lane_emb,
           lane_bias, W1, b1, W2, b2, W3, b3, W4, b4, W5, b5):
    idx3 = inputs[:, 0:3].astype(jnp.int32)
    idxw = (idx3.reshape(_NCHUNK, _CB, 3).transpose(0, 2, 1)
            .reshape(_NCHUNK, 3, _NW, _IPR, 128)
            .transpose(0, 2, 1, 3, 4).reshape(_NCHUNK, _NW, 3 * _IPR, 128))

    t0 = speed_emb.reshape(-1, 128)
    t1 = oneway_emb.reshape(-1, 128)
    t2 = lane_emb.reshape(-1, 128)

    def padbias(bias):
        v = bias.reshape(-1)
        n = v.shape[0]
        pad = (-n) % 128
        return jnp.pad(v, (0, pad)).reshape(-1, 128)

    b0 = padbias(speed_bias)
    b1v = padbias(oneway_bias)
    b2v = padbias(lane_bias)

    w1s = jnp.sum(W1[:3 * EMB], axis=0, keepdims=True)
    a1 = jnp.concatenate([W1[0:EMB], w1s, W1[EMB:2 * EMB], w1s,
                          W1[2 * EMB:3 * EMB], w1s], axis=0)
    w1n = W1[3 * EMB:]

    outs = []
    for q in range(_NCHUNK):
        g = _sc_gather(idxw[q], t0, t1, t2, b0, b1v, b2v)
        outs.append(_mlp(
            lax.slice_in_dim(inputs, q * _CB, (q + 1) * _CB), g, a1, w1n,
            b1.reshape(1, -1), W2, b2.reshape(1, -1),
            W3, b3.reshape(1, -1), W4, b4.reshape(1, -1),
            W5, b5.reshape(1, -1), block_b=min(4096, _CB)))
    return jnp.concatenate(outs, axis=0)


# speedup vs baseline: 1.0554x; 1.0554x over previous
import functools

import jax
import jax.numpy as jnp
from jax import lax
from jax.experimental import pallas as pl
from jax.experimental.pallas import tpu as pltpu
from jax.experimental.pallas import tpu_sc as plsc

BATCH = 16384
EMB = 16
NUM_NUM = 16
UNITS = 128
GCOLS = 3 * (EMB + 1)

_NC = 2
_NS = 16
_NW = _NC * _NS
_L = 16
_NCHUNK = 1
_CB = BATCH // _NCHUNK
_BPW = _CB // _NW
_IPR = _BPW // 128


def _sc_gather_body(idx_hbm, t0, t1, t2, b0, b1, b2, g_hbm,
                    idx_v, t0v, t1v, t2v, b0v, b1v, b2v, g_v):
    wid = lax.axis_index("s") * _NC + lax.axis_index("c")
    pltpu.sync_copy(idx_hbm.at[wid], idx_v)
    pltpu.sync_copy(t0, t0v)
    pltpu.sync_copy(t1, t1v)
    pltpu.sync_copy(t2, t2v)
    pltpu.sync_copy(b0, b0v)
    pltpu.sync_copy(b1, b1v)
    pltpu.sync_copy(b2, b2v)

    tables = ((t0v, b0v, 0), (t1v, b1v, EMB + 1), (t2v, b2v, 2 * (EMB + 1)))
    for kk in range(_IPR):
        for c in range(8):
            k = kk * 8 + c
            for t, (tv, bv, row0) in enumerate(tables):
                iv = idx_v[t * _IPR + kk, pl.ds(c * _L, _L)]
                lin = iv * EMB
                for j in range(EMB):
                    lj = lin + j
                    vals = plsc.load_gather(
                        tv, [lax.shift_right_logical(lj, 7),
                             lax.bitwise_and(lj, 127)])
                    g_v[row0 + j, pl.ds(k * _L, _L)] = vals
                bvals = plsc.load_gather(
                    bv, [lax.shift_right_logical(iv, 7),
                         lax.bitwise_and(iv, 127)])
                g_v[row0 + EMB, pl.ds(k * _L, _L)] = bvals

    pltpu.sync_copy(g_v, g_hbm.at[:, pl.ds(wid * _BPW, _BPW)])


def _sc_gather(idxw, t0, t1, t2, b0, b1, b2):
    mesh = plsc.VectorSubcoreMesh(core_axis_name="c", subcore_axis_name="s")
    f = functools.partial(
        pl.kernel,
        mesh=mesh,
        out_type=jax.ShapeDtypeStruct((GCOLS, _CB), jnp.float32),
        scratch_types=[
            pltpu.VMEM((3 * _IPR, 128), jnp.int32),
            pltpu.VMEM(t0.shape, jnp.float32),
            pltpu.VMEM(t1.shape, jnp.float32),
            pltpu.VMEM(t2.shape, jnp.float32),
            pltpu.VMEM(b0.shape, jnp.float32),
            pltpu.VMEM(b1.shape, jnp.float32),
            pltpu.VMEM(b2.shape, jnp.float32),
            pltpu.VMEM((GCOLS, _BPW), jnp.float32),
        ],
        compiler_params=pltpu.CompilerParams(needs_layout_passes=False),
    )(_sc_gather_body)
    return f(idxw, t0, t1, t2, b0, b1, b2)


def _mlp_body(inp_ref, g_ref, a1, w1n, c1, w2, c2, w3, c3, w4, c4, w5, c5,
              out_ref):
    dot = functools.partial(jnp.dot, preferred_element_type=jnp.float32)
    num = inp_ref[:, 3:3 + NUM_NUM]
    h = lax.dot_general(g_ref[...], a1[...], (((0,), (0,)), ((), ())),
                        preferred_element_type=jnp.float32)
    h = h + dot(num, w1n[...]) + c1[...]
    h = jnp.maximum(h, 0.0)
    h = jnp.maximum(dot(h, w2[...]) + c2[...], 0.0)
    h = jnp.maximum(dot(h, w3[...]) + c3[...], 0.0)
    h = jnp.maximum(dot(h, w4[...]) + c4[...], 0.0)
    logits = dot(h, w5[...]) + c5[...]
    m = jnp.max(logits, axis=-1, keepdims=True)
    e = jnp.exp(logits - m)
    out_ref[...] = e / jnp.sum(e, axis=-1, keepdims=True)


def _full(shape):
    return pl.BlockSpec(shape, lambda i: (0, 0))


def _mlp(inputs, g, a1, w1n, c1, w2, c2, w3, c3, w4, c4, w5, c5, block_b):
    nlab = w5.shape[1]
    nb = inputs.shape[0]
    grid = (nb // block_b,)
    in_specs = [
        pl.BlockSpec((block_b, inputs.shape[1]), lambda i: (i, 0)),
        pl.BlockSpec((GCOLS, block_b), lambda i: (0, i)),
        _full(a1.shape), _full(w1n.shape), _full(c1.shape),
        _full(w2.shape), _full(c2.shape), _full(w3.shape), _full(c3.shape),
        _full(w4.shape), _full(c4.shape), _full(w5.shape), _full(c5.shape),
    ]
    return pl.pallas_call(
        _mlp_body,
        grid=grid,
        in_specs=in_specs,
        out_specs=pl.BlockSpec((block_b, nlab), lambda i: (i, 0)),
        out_shape=jax.ShapeDtypeStruct((nb, nlab), jnp.float32),
        compiler_params=pltpu.CompilerParams(
            dimension_semantics=("arbitrary",)),
    )(inputs, g, a1, w1n, c1, w2, c2, w3, c3, w4, c4, w5, c5)


def kernel(inputs, speed_emb, speed_bias, oneway_emb, oneway_bias, lane_emb,
           lane_bias, W1, b1, W2, b2, W3, b3, W4, b4, W5, b5):
    idx3 = inputs[:, 0:3].astype(jnp.int32)
    idxw = (idx3.reshape(_NCHUNK, _CB, 3).transpose(0, 2, 1)
            .reshape(_NCHUNK, 3, _NW, _IPR, 128)
            .transpose(0, 2, 1, 3, 4).reshape(_NCHUNK, _NW, 3 * _IPR, 128))

    t0 = speed_emb.reshape(-1, 128)
    t1 = oneway_emb.reshape(-1, 128)
    t2 = lane_emb.reshape(-1, 128)

    def padbias(bias):
        v = bias.reshape(-1)
        n = v.shape[0]
        pad = (-n) % 128
        return jnp.pad(v, (0, pad)).reshape(-1, 128)

    b0 = padbias(speed_bias)
    b1v = padbias(oneway_bias)
    b2v = padbias(lane_bias)

    w1s = jnp.sum(W1[:3 * EMB], axis=0, keepdims=True)
    a1 = jnp.concatenate([W1[0:EMB], w1s, W1[EMB:2 * EMB], w1s,
                          W1[2 * EMB:3 * EMB], w1s], axis=0)
    w1n = W1[3 * EMB:]

    outs = []
    for q in range(_NCHUNK):
        g = _sc_gather(idxw[q], t0, t1, t2, b0, b1v, b2v)
        outs.append(_mlp(
            lax.slice_in_dim(inputs, q * _CB, (q + 1) * _CB), g, a1, w1n,
            b1.reshape(1, -1), W2, b2.reshape(1, -1),
            W3, b3.reshape(1, -1), W4, b4.reshape(1, -1),
            W5, b5.reshape(1, -1), block_b=min(4096, _CB)))
    return jnp.concatenate(outs, axis=0)


# speedup vs baseline: 1.1307x; 1.0714x over previous
import functools

import jax
import jax.numpy as jnp
from jax import lax
from jax.experimental import pallas as pl
from jax.experimental.pallas import tpu as pltpu
from jax.experimental.pallas import tpu_sc as plsc

BATCH = 16384
EMB = 16
NUM_NUM = 16
UNITS = 128
GCOLS = 3 * (EMB + 1)

_NC = 2
_NS = 16
_NW = _NC * _NS
_L = 16
_NCHUNK = 1
_CB = BATCH // _NCHUNK
_BPW = _CB // _NW
_IPR = _BPW // 128

_N0, _N1, _N2 = 1000, 8, 16
_OFF_E = (0, _N0 * EMB, (_N0 + _N1) * EMB)
_OFF_B = ((_N0 + _N1 + _N2) * EMB,
          (_N0 + _N1 + _N2) * EMB + _N0,
          (_N0 + _N1 + _N2) * EMB + _N0 + _N1)
_TROWS = ((_N0 + _N1 + _N2) * (EMB + 1) + 127) // 128


def _sc_gather_body(idx_hbm, tt, g_hbm, idx_v, ttv, g_v):
    wid = lax.axis_index("s") * _NC + lax.axis_index("c")
    pltpu.sync_copy(idx_hbm.at[wid], idx_v)
    pltpu.sync_copy(tt, ttv)

    for kk in range(_IPR):
        for c in range(8):
            k = kk * 8 + c
            for t in range(3):
                row0 = t * (EMB + 1)
                iv = idx_v[t * _IPR + kk, pl.ds(c * _L, _L)]
                lin = iv * EMB + _OFF_E[t]
                for j in range(EMB):
                    lj = lin + j
                    vals = plsc.load_gather(
                        ttv, [lax.shift_right_logical(lj, 7),
                              lax.bitwise_and(lj, 127)])
                    g_v[row0 + j, pl.ds(k * _L, _L)] = vals
                fb = iv + _OFF_B[t]
                bvals = plsc.load_gather(
                    ttv, [lax.shift_right_logical(fb, 7),
                          lax.bitwise_and(fb, 127)])
                g_v[row0 + EMB, pl.ds(k * _L, _L)] = bvals

    pltpu.sync_copy(g_v, g_hbm.at[:, pl.ds(wid * _BPW, _BPW)])


def _sc_gather(idxw, tt):
    mesh = plsc.VectorSubcoreMesh(core_axis_name="c", subcore_axis_name="s")
    f = functools.partial(
        pl.kernel,
        mesh=mesh,
        out_type=jax.ShapeDtypeStruct((GCOLS, _CB), jnp.float32),
        scratch_types=[
            pltpu.VMEM((3 * _IPR, 128), jnp.int32),
            pltpu.VMEM((_TROWS, 128), jnp.float32),
            pltpu.VMEM((GCOLS, _BPW), jnp.float32),
        ],
        compiler_params=pltpu.CompilerParams(needs_layout_passes=False),
    )(_sc_gather_body)
    return f(idxw, tt)


def _mlp_body(inp_ref, g_ref, a1, w1n, c1, w2, c2, w3, c3, w4, c4, w5, c5,
              out_ref):
    dot = functools.partial(jnp.dot, preferred_element_type=jnp.float32)
    num = inp_ref[:, 3:3 + NUM_NUM]
    h = lax.dot_general(g_ref[...], a1[...], (((0,), (0,)), ((), ())),
                        preferred_element_type=jnp.float32)
    h = h + dot(num, w1n[...]) + c1[...]
    h = jnp.maximum(h, 0.0)
    h = jnp.maximum(dot(h, w2[...]) + c2[...], 0.0)
    h = jnp.maximum(dot(h, w3[...]) + c3[...], 0.0)
    h = jnp.maximum(dot(h, w4[...]) + c4[...], 0.0)
    logits = dot(h, w5[...]) + c5[...]
    m = jnp.max(logits, axis=-1, keepdims=True)
    e = jnp.exp(logits - m)
    out_ref[...] = e / jnp.sum(e, axis=-1, keepdims=True)


def _full(shape):
    return pl.BlockSpec(shape, lambda i: (0, 0))


def _mlp(inputs, g, a1, w1n, c1, w2, c2, w3, c3, w4, c4, w5, c5, block_b):
    nlab = w5.shape[1]
    nb = inputs.shape[0]
    grid = (nb // block_b,)
    in_specs = [
        pl.BlockSpec((block_b, inputs.shape[1]), lambda i: (i, 0)),
        pl.BlockSpec((GCOLS, block_b), lambda i: (0, i)),
        _full(a1.shape), _full(w1n.shape), _full(c1.shape),
        _full(w2.shape), _full(c2.shape), _full(w3.shape), _full(c3.shape),
        _full(w4.shape), _full(c4.shape), _full(w5.shape), _full(c5.shape),
    ]
    return pl.pallas_call(
        _mlp_body,
        grid=grid,
        in_specs=in_specs,
        out_specs=pl.BlockSpec((block_b, nlab), lambda i: (i, 0)),
        out_shape=jax.ShapeDtypeStruct((nb, nlab), jnp.float32),
        compiler_params=pltpu.CompilerParams(
            dimension_semantics=("arbitrary",)),
    )(inputs, g, a1, w1n, c1, w2, c2, w3, c3, w4, c4, w5, c5)


def kernel(inputs, speed_emb, speed_bias, oneway_emb, oneway_bias, lane_emb,
           lane_bias, W1, b1, W2, b2, W3, b3, W4, b4, W5, b5):
    idx3 = inputs[:, 0:3].astype(jnp.int32)
    idxw = (idx3.reshape(_NCHUNK, _CB, 3).transpose(0, 2, 1)
            .reshape(_NCHUNK, 3, _NW, _IPR, 128)
            .transpose(0, 2, 1, 3, 4).reshape(_NCHUNK, _NW, 3 * _IPR, 128))

    flat = jnp.concatenate([
        speed_emb.reshape(-1), oneway_emb.reshape(-1), lane_emb.reshape(-1),
        speed_bias.reshape(-1), oneway_bias.reshape(-1),
        lane_bias.reshape(-1)])
    tt = jnp.pad(flat, (0, _TROWS * 128 - flat.shape[0])).reshape(_TROWS, 128)

    w1s = jnp.sum(W1[:3 * EMB], axis=0, keepdims=True)
    a1 = jnp.concatenate([W1[0:EMB], w1s, W1[EMB:2 * EMB], w1s,
                          W1[2 * EMB:3 * EMB], w1s], axis=0)
    w1n = W1[3 * EMB:]

    outs = []
    for q in range(_NCHUNK):
        g = _sc_gather(idxw[q], tt)
        outs.append(_mlp(
            lax.slice_in_dim(inputs, q * _CB, (q + 1) * _CB), g, a1, w1n,
            b1.reshape(1, -1), W2, b2.reshape(1, -1),
            W3, b3.reshape(1, -1), W4, b4.reshape(1, -1),
            W5, b5.reshape(1, -1), block_b=min(4096, _CB)))
    return jnp.concatenate(outs, axis=0)


# speedup vs baseline: 1.1322x; 1.0013x over previous
import functools

import jax
import jax.numpy as jnp
from jax import lax
from jax.experimental import pallas as pl
from jax.experimental.pallas import tpu as pltpu
from jax.experimental.pallas import tpu_sc as plsc

BATCH = 16384
EMB = 16
NUM_NUM = 16
UNITS = 128
GCOLS = 3 * (EMB + 1)

_NC = 2
_NS = 16
_NW = _NC * _NS
_L = 16
_NCHUNK = 1
_CB = BATCH // _NCHUNK
_BPW = _CB // _NW
_IPR = _BPW // 128

_N0, _N1, _N2 = 1000, 8, 16
_OFF_E = (0, _N0 * EMB, (_N0 + _N1) * EMB)
_OFF_B = ((_N0 + _N1 + _N2) * EMB,
          (_N0 + _N1 + _N2) * EMB + _N0,
          (_N0 + _N1 + _N2) * EMB + _N0 + _N1)
_TROWS = ((_N0 + _N1 + _N2) * (EMB + 1) + 127) // 128


def _sc_gather_body(idx_hbm, tt, g_hbm, idx_v, ttv, g_v):
    wid = lax.axis_index("s") * _NC + lax.axis_index("c")
    pltpu.sync_copy(idx_hbm.at[wid], idx_v)
    pltpu.sync_copy(tt, ttv)

    for kk in range(_IPR):
        for c in range(8):
            k = kk * 8 + c
            for t in range(3):
                row0 = t * (EMB + 1)
                iv = idx_v[t * _IPR + kk, pl.ds(c * _L, _L)]
                lin = iv * EMB + _OFF_E[t]
                hi = lax.shift_right_logical(lin, 7)
                lo = lax.bitwise_and(lin, 127)
                for j in range(EMB):
                    vals = plsc.load_gather(ttv, [hi, lo + j])
                    g_v[row0 + j, pl.ds(k * _L, _L)] = vals
                fb = iv + _OFF_B[t]
                bvals = plsc.load_gather(
                    ttv, [lax.shift_right_logical(fb, 7),
                          lax.bitwise_and(fb, 127)])
                g_v[row0 + EMB, pl.ds(k * _L, _L)] = bvals

    pltpu.sync_copy(g_v, g_hbm.at[:, pl.ds(wid * _BPW, _BPW)])


def _sc_gather(idxw, tt):
    mesh = plsc.VectorSubcoreMesh(core_axis_name="c", subcore_axis_name="s")
    f = functools.partial(
        pl.kernel,
        mesh=mesh,
        out_type=jax.ShapeDtypeStruct((GCOLS, _CB), jnp.float32),
        scratch_types=[
            pltpu.VMEM((3 * _IPR, 128), jnp.int32),
            pltpu.VMEM((_TROWS, 128), jnp.float32),
            pltpu.VMEM((GCOLS, _BPW), jnp.float32),
        ],
        compiler_params=pltpu.CompilerParams(needs_layout_passes=False),
    )(_sc_gather_body)
    return f(idxw, tt)


def _mlp_body(inp_ref, g_ref, a1, w1n, c1, w2, c2, w3, c3, w4, c4, w5, c5,
              out_ref):
    dot = functools.partial(jnp.dot, preferred_element_type=jnp.float32)
    num = inp_ref[:, 3:3 + NUM_NUM]
    h = lax.dot_general(g_ref[...], a1[...], (((0,), (0,)), ((), ())),
                        preferred_element_type=jnp.float32)
    h = h + dot(num, w1n[...]) + c1[...]
    h = jnp.maximum(h, 0.0)
    h = jnp.maximum(dot(h, w2[...]) + c2[...], 0.0)
    h = jnp.maximum(dot(h, w3[...]) + c3[...], 0.0)
    h = jnp.maximum(dot(h, w4[...]) + c4[...], 0.0)
    logits = dot(h, w5[...]) + c5[...]
    m = jnp.max(logits, axis=-1, keepdims=True)
    e = jnp.exp(logits - m)
    out_ref[...] = e / jnp.sum(e, axis=-1, keepdims=True)


def _full(shape):
    return pl.BlockSpec(shape, lambda i: (0, 0))


def _mlp(inputs, g, a1, w1n, c1, w2, c2, w3, c3, w4, c4, w5, c5, block_b):
    nlab = w5.shape[1]
    nb = inputs.shape[0]
    grid = (nb // block_b,)
    in_specs = [
        pl.BlockSpec((block_b, inputs.shape[1]), lambda i: (i, 0)),
        pl.BlockSpec((GCOLS, block_b), lambda i: (0, i)),
        _full(a1.shape), _full(w1n.shape), _full(c1.shape),
        _full(w2.shape), _full(c2.shape), _full(w3.shape), _full(c3.shape),
        _full(w4.shape), _full(c4.shape), _full(w5.shape), _full(c5.shape),
    ]
    return pl.pallas_call(
        _mlp_body,
        grid=grid,
        in_specs=in_specs,
        out_specs=pl.BlockSpec((block_b, nlab), lambda i: (i, 0)),
        out_shape=jax.ShapeDtypeStruct((nb, nlab), jnp.float32),
        compiler_params=pltpu.CompilerParams(
            dimension_semantics=("arbitrary",)),
    )(inputs, g, a1, w1n, c1, w2, c2, w3, c3, w4, c4, w5, c5)


def kernel(inputs, speed_emb, speed_bias, oneway_emb, oneway_bias, lane_emb,
           lane_bias, W1, b1, W2, b2, W3, b3, W4, b4, W5, b5):
    idx3 = inputs[:, 0:3].astype(jnp.int32)
    idxw = (idx3.reshape(_NCHUNK, _CB, 3).transpose(0, 2, 1)
            .reshape(_NCHUNK, 3, _NW, _IPR, 128)
            .transpose(0, 2, 1, 3, 4).reshape(_NCHUNK, _NW, 3 * _IPR, 128))

    flat = jnp.concatenate([
        speed_emb.reshape(-1), oneway_emb.reshape(-1), lane_emb.reshape(-1),
        speed_bias.reshape(-1), oneway_bias.reshape(-1),
        lane_bias.reshape(-1)])
    tt = jnp.pad(flat, (0, _TROWS * 128 - flat.shape[0])).reshape(_TROWS, 128)

    w1s = jnp.sum(W1[:3 * EMB], axis=0, keepdims=True)
    a1 = jnp.concatenate([W1[0:EMB], w1s, W1[EMB:2 * EMB], w1s,
                          W1[2 * EMB:3 * EMB], w1s], axis=0)
    w1n = W1[3 * EMB:]

    outs = []
    for q in range(_NCHUNK):
        g = _sc_gather(idxw[q], tt)
        outs.append(_mlp(
            lax.slice_in_dim(inputs, q * _CB, (q + 1) * _CB), g, a1, w1n,
            b1.reshape(1, -1), W2, b2.reshape(1, -1),
            W3, b3.reshape(1, -1), W4, b4.reshape(1, -1),
            W5, b5.reshape(1, -1), block_b=min(4096, _CB)))
    return jnp.concatenate(outs, axis=0)


# speedup vs baseline: 1.1670x; 1.0308x over previous
import functools

import jax
import jax.numpy as jnp
from jax import lax
from jax.experimental import pallas as pl
from jax.experimental.pallas import tpu as pltpu
from jax.experimental.pallas import tpu_sc as plsc

BATCH = 16384
EMB = 16
NUM_NUM = 16
UNITS = 128
GCOLS = 3 * (EMB + 1)

_NC = 2
_NS = 16
_NW = _NC * _NS
_L = 16
_NCHUNK = 1
_CB = BATCH // _NCHUNK
_BPW = _CB // _NW
_IPR = _BPW // 128

_N0, _N1, _N2 = 1000, 8, 16
_OFF_E = (0, _N0 * EMB, (_N0 + _N1) * EMB)
_OFF_B = ((_N0 + _N1 + _N2) * EMB,
          (_N0 + _N1 + _N2) * EMB + _N0,
          (_N0 + _N1 + _N2) * EMB + _N0 + _N1)
_TROWS = ((_N0 + _N1 + _N2) * (EMB + 1) + 127) // 128


def _sc_gather_body(idx_hbm, tt, g_hbm, idx_v, ttv, g_v):
    wid = lax.axis_index("s") * _NC + lax.axis_index("c")
    pltpu.sync_copy(idx_hbm.at[wid], idx_v)
    pltpu.sync_copy(tt, ttv)

    def chunk(k, carry):
        kk = lax.shift_right_logical(k, 3)
        off = lax.bitwise_and(k, 7) * _L
        for t in range(3):
            row0 = t * (EMB + 1)
            iv = idx_v[t * _IPR + kk, pl.ds(off, _L)]
            lin = iv * EMB + _OFF_E[t]
            hi = lax.shift_right_logical(lin, 7)
            lo = lax.bitwise_and(lin, 127)
            for j in range(EMB):
                vals = plsc.load_gather(ttv, [hi, lo + j])
                g_v[row0 + j, pl.ds(k * _L, _L)] = vals
            fb = iv + _OFF_B[t]
            bvals = plsc.load_gather(
                ttv, [lax.shift_right_logical(fb, 7),
                      lax.bitwise_and(fb, 127)])
            g_v[row0 + EMB, pl.ds(k * _L, _L)] = bvals
        return carry

    lax.fori_loop(0, _BPW // _L, chunk, 0)

    pltpu.sync_copy(g_v, g_hbm.at[:, pl.ds(wid * _BPW, _BPW)])


def _sc_gather(idxw, tt):
    mesh = plsc.VectorSubcoreMesh(core_axis_name="c", subcore_axis_name="s")
    f = functools.partial(
        pl.kernel,
        mesh=mesh,
        out_type=jax.ShapeDtypeStruct((GCOLS, _CB), jnp.float32),
        scratch_types=[
            pltpu.VMEM((3 * _IPR, 128), jnp.int32),
            pltpu.VMEM((_TROWS, 128), jnp.float32),
            pltpu.VMEM((GCOLS, _BPW), jnp.float32),
        ],
        compiler_params=pltpu.CompilerParams(needs_layout_passes=False),
    )(_sc_gather_body)
    return f(idxw, tt)


def _mlp_body(inp_ref, g_ref, a1, w1n, c1, w2, c2, w3, c3, w4, c4, w5, c5,
              out_ref):
    dot = functools.partial(jnp.dot, preferred_element_type=jnp.float32)
    num = inp_ref[:, 3:3 + NUM_NUM]
    h = lax.dot_general(g_ref[...], a1[...], (((0,), (0,)), ((), ())),
                        preferred_element_type=jnp.float32)
    h = h + dot(num, w1n[...]) + c1[...]
    h = jnp.maximum(h, 0.0)
    h = jnp.maximum(dot(h, w2[...]) + c2[...], 0.0)
    h = jnp.maximum(dot(h, w3[...]) + c3[...], 0.0)
    h = jnp.maximum(dot(h, w4[...]) + c4[...], 0.0)
    logits = dot(h, w5[...]) + c5[...]
    m = jnp.max(logits, axis=-1, keepdims=True)
    e = jnp.exp(logits - m)
    out_ref[...] = e / jnp.sum(e, axis=-1, keepdims=True)


def _full(shape):
    return pl.BlockSpec(shape, lambda i: (0, 0))


def _mlp(inputs, g, a1, w1n, c1, w2, c2, w3, c3, w4, c4, w5, c5, block_b):
    nlab = w5.shape[1]
    nb = inputs.shape[0]
    grid = (nb // block_b,)
    in_specs = [
        pl.BlockSpec((block_b, inputs.shape[1]), lambda i: (i, 0)),
        pl.BlockSpec((GCOLS, block_b), lambda i: (0, i)),
        _full(a1.shape), _full(w1n.shape), _full(c1.shape),
        _full(w2.shape), _full(c2.shape), _full(w3.shape), _full(c3.shape),
        _full(w4.shape), _full(c4.shape), _full(w5.shape), _full(c5.shape),
    ]
    return pl.pallas_call(
        _mlp_body,
        grid=grid,
        in_specs=in_specs,
        out_specs=pl.BlockSpec((block_b, nlab), lambda i: (i, 0)),
        out_shape=jax.ShapeDtypeStruct((nb, nlab), jnp.float32),
        compiler_params=pltpu.CompilerParams(
            dimension_semantics=("arbitrary",)),
    )(inputs, g, a1, w1n, c1, w2, c2, w3, c3, w4, c4, w5, c5)


def kernel(inputs, speed_emb, speed_bias, oneway_emb, oneway_bias, lane_emb,
           lane_bias, W1, b1, W2, b2, W3, b3, W4, b4, W5, b5):
    idx3 = inputs[:, 0:3].astype(jnp.int32)
    idxw = (idx3.reshape(_NCHUNK, _CB, 3).transpose(0, 2, 1)
            .reshape(_NCHUNK, 3, _NW, _IPR, 128)
            .transpose(0, 2, 1, 3, 4).reshape(_NCHUNK, _NW, 3 * _IPR, 128))

    flat = jnp.concatenate([
        speed_emb.reshape(-1), oneway_emb.reshape(-1), lane_emb.reshape(-1),
        speed_bias.reshape(-1), oneway_bias.reshape(-1),
        lane_bias.reshape(-1)])
    tt = jnp.pad(flat, (0, _TROWS * 128 - flat.shape[0])).reshape(_TROWS, 128)

    w1s = jnp.sum(W1[:3 * EMB], axis=0, keepdims=True)
    a1 = jnp.concatenate([W1[0:EMB], w1s, W1[EMB:2 * EMB], w1s,
                          W1[2 * EMB:3 * EMB], w1s], axis=0)
    w1n = W1[3 * EMB:]

    outs = []
    for q in range(_NCHUNK):
        g = _sc_gather(idxw[q], tt)
        outs.append(_mlp(
            lax.slice_in_dim(inputs, q * _CB, (q + 1) * _CB), g, a1, w1n,
            b1.reshape(1, -1), W2, b2.reshape(1, -1),
            W3, b3.reshape(1, -1), W4, b4.reshape(1, -1),
            W5, b5.reshape(1, -1), block_b=min(4096, _CB)))
    return jnp.concatenate(outs, axis=0)


# speedup vs baseline: 1.1700x; 1.0025x over previous
import functools

import jax
import jax.numpy as jnp
from jax import lax
from jax.experimental import pallas as pl
from jax.experimental.pallas import tpu as pltpu
from jax.experimental.pallas import tpu_sc as plsc

BATCH = 16384
EMB = 16
NUM_NUM = 16
UNITS = 128
GCOLS = 3 * (EMB + 1)

_NC = 2
_NS = 16
_NW = _NC * _NS
_L = 16
_NCHUNK = 1
_CB = BATCH // _NCHUNK
_BPW = _CB // _NW
_IPR = _BPW // 128

_N0, _N1, _N2 = 1000, 8, 16
_OFF_E = (0, _N0 * EMB, (_N0 + _N1) * EMB)
_OFF_B = ((_N0 + _N1 + _N2) * EMB,
          (_N0 + _N1 + _N2) * EMB + _N0,
          (_N0 + _N1 + _N2) * EMB + _N0 + _N1)
_TROWS = ((_N0 + _N1 + _N2) * (EMB + 1) + 127) // 128


def _sc_gather_body(idx_hbm, tt, g_hbm, idx_v, ttv, g_v):
    wid = lax.axis_index("s") * _NC + lax.axis_index("c")
    pltpu.sync_copy(idx_hbm.at[wid], idx_v)
    pltpu.sync_copy(tt, ttv)

    def chunk(k, carry):
        kk = lax.shift_right_logical(k, 3)
        off = lax.bitwise_and(k, 7) * _L
        for t in range(3):
            row0 = t * EMB
            iv = idx_v[t * _IPR + kk, pl.ds(off, _L)]
            lin = iv * EMB + _OFF_E[t]
            hi = lax.shift_right_logical(lin, 7)
            lo = lax.bitwise_and(lin, 127)
            for j in range(EMB):
                vals = plsc.load_gather(ttv, [hi, lo + j])
                g_v[row0 + j, pl.ds(k * _L, _L)] = vals
            fb = iv + _OFF_B[t]
            bvals = plsc.load_gather(
                ttv, [lax.shift_right_logical(fb, 7),
                      lax.bitwise_and(fb, 127)])
            g_v[3 * EMB + t, pl.ds(k * _L, _L)] = bvals
        return carry

    lax.fori_loop(0, _BPW // _L, chunk, 0)

    pltpu.sync_copy(g_v, g_hbm.at[:, pl.ds(wid * _BPW, _BPW)])


def _sc_gather(idxw, tt):
    mesh = plsc.VectorSubcoreMesh(core_axis_name="c", subcore_axis_name="s")
    f = functools.partial(
        pl.kernel,
        mesh=mesh,
        out_type=jax.ShapeDtypeStruct((GCOLS, _CB), jnp.float32),
        scratch_types=[
            pltpu.VMEM((3 * _IPR, 128), jnp.int32),
            pltpu.VMEM((_TROWS, 128), jnp.float32),
            pltpu.VMEM((GCOLS, _BPW), jnp.float32),
        ],
        compiler_params=pltpu.CompilerParams(needs_layout_passes=False),
    )(_sc_gather_body)
    return f(idxw, tt)


def _mlp_body(inp_ref, g_ref, w1, c1, w2, c2, w3, c3, w4, c4, w5, c5,
              out_ref):
    dot = functools.partial(jnp.dot, preferred_element_type=jnp.float32)
    num = inp_ref[:, 3:3 + NUM_NUM]
    w1e = w1[0:3 * EMB, :]
    w1s = jnp.sum(w1e, axis=0, keepdims=True)
    a1 = jnp.concatenate([w1e, w1s, w1s, w1s], axis=0)
    h = lax.dot_general(g_ref[...], a1, (((0,), (0,)), ((), ())),
                        preferred_element_type=jnp.float32)
    h = h + dot(num, w1[3 * EMB:, :]) + c1[...]
    h = jnp.maximum(h, 0.0)
    h = jnp.maximum(dot(h, w2[...]) + c2[...], 0.0)
    h = jnp.maximum(dot(h, w3[...]) + c3[...], 0.0)
    h = jnp.maximum(dot(h, w4[...]) + c4[...], 0.0)
    logits = dot(h, w5[...]) + c5[...]
    m = jnp.max(logits, axis=-1, keepdims=True)
    e = jnp.exp(logits - m)
    out_ref[...] = e / jnp.sum(e, axis=-1, keepdims=True)


def _full(shape):
    return pl.BlockSpec(shape, lambda i: (0, 0))


def _mlp(inputs, g, w1, c1, w2, c2, w3, c3, w4, c4, w5, c5, block_b):
    nlab = w5.shape[1]
    nb = inputs.shape[0]
    grid = (nb // block_b,)
    in_specs = [
        pl.BlockSpec((block_b, inputs.shape[1]), lambda i: (i, 0)),
        pl.BlockSpec((GCOLS, block_b), lambda i: (0, i)),
        _full(w1.shape), _full(c1.shape),
        _full(w2.shape), _full(c2.shape), _full(w3.shape), _full(c3.shape),
        _full(w4.shape), _full(c4.shape), _full(w5.shape), _full(c5.shape),
    ]
    return pl.pallas_call(
        _mlp_body,
        grid=grid,
        in_specs=in_specs,
        out_specs=pl.BlockSpec((block_b, nlab), lambda i: (i, 0)),
        out_shape=jax.ShapeDtypeStruct((nb, nlab), jnp.float32),
        compiler_params=pltpu.CompilerParams(
            dimension_semantics=("arbitrary",)),
    )(inputs, g, w1, c1, w2, c2, w3, c3, w4, c4, w5, c5)


def kernel(inputs, speed_emb, speed_bias, oneway_emb, oneway_bias, lane_emb,
           lane_bias, W1, b1, W2, b2, W3, b3, W4, b4, W5, b5):
    idx3 = inputs[:, 0:3].astype(jnp.int32)
    idxw = (idx3.reshape(_NCHUNK, _CB, 3).transpose(0, 2, 1)
            .reshape(_NCHUNK, 3, _NW, _IPR, 128)
            .transpose(0, 2, 1, 3, 4).reshape(_NCHUNK, _NW, 3 * _IPR, 128))

    flat = jnp.concatenate([
        speed_emb.reshape(-1), oneway_emb.reshape(-1), lane_emb.reshape(-1),
        speed_bias.reshape(-1), oneway_bias.reshape(-1),
        lane_bias.reshape(-1)])
    tt = jnp.pad(flat, (0, _TROWS * 128 - flat.shape[0])).reshape(_TROWS, 128)

    outs = []
    for q in range(_NCHUNK):
        g = _sc_gather(idxw[q], tt)
        outs.append(_mlp(
            lax.slice_in_dim(inputs, q * _CB, (q + 1) * _CB), g, W1,
            b1.reshape(1, -1), W2, b2.reshape(1, -1),
            W3, b3.reshape(1, -1), W4, b4.reshape(1, -1),
            W5, b5.reshape(1, -1), block_b=min(4096, _CB)))
    return outs[0] if _NCHUNK == 1 else jnp.concatenate(outs, axis=0)
